# baseline (speedup 1.0000x reference)
import jax
import jax.numpy as jnp
from jax.experimental import pallas as pl
from jax.experimental.pallas import tpu as pltpu

VOCAB = 1024
E = 256
K = 8
T = 4
H = 4 * E
VB = 1024
NV = VOCAB // VB
EPS = 1e-5


def _layernorm(x):
    mu = jnp.mean(x, axis=-1, keepdims=True)
    var = jnp.mean((x - mu) ** 2, axis=-1, keepdims=True)
    return (x - mu) * jax.lax.rsqrt(var + EPS)


def _mlp_chain(x, w1_ref, b1_ref, w2_ref, b2_ref, w3_ref, b3_ref,
               w4_ref, b4_ref):
    h = jnp.maximum(
        jnp.dot(x, w1_ref[...], preferred_element_type=jnp.float32)
        + b1_ref[...], 0.0)
    h = jnp.maximum(
        jnp.dot(h, w2_ref[...], preferred_element_type=jnp.float32)
        + b2_ref[...], 0.0)
    h = jnp.maximum(
        jnp.dot(h, w3_ref[...], preferred_element_type=jnp.float32)
        + b3_ref[...], 0.0)
    m = (jnp.dot(h, w4_ref[...], preferred_element_type=jnp.float32)
         + b4_ref[...])
    return _layernorm(m)


def _fused_kernel(emb_ref, z_ref, w1_ref, b1_ref, w2_ref, b2_ref,
                  w3_ref, b3_ref, w4_ref, b4_ref, tok_ref, zq_ref, mem_s):
    v = pl.program_id(0)
    for t in range(T):
        x = emb_ref[:, t * E:(t + 1) * E]
        mem_s[t, pl.ds(v * VB, VB), :] = _mlp_chain(
            x, w1_ref, b1_ref, w2_ref, b2_ref, w3_ref, b3_ref,
            w4_ref, b4_ref)

    @pl.when(v == NV - 1)
    def _attention():
        toks = []
        for t in range(T):
            q = z_ref[:, t * E:(t + 1) * E]
            qn = _layernorm(q) * (E ** -0.5)
            memt = mem_s[t]
            s = jax.lax.dot_general(
                qn, memt, (((1,), (1,)), ((), ())),
                preferred_element_type=jnp.float32)
            mx = jnp.max(s, axis=-1, keepdims=True)
            e = jnp.exp(s - mx)
            rcp = 1.0 / jnp.sum(e, axis=-1, keepdims=True)
            idx = jax.lax.broadcasted_iota(jnp.int32, s.shape, 1)
            toks.append(jnp.min(jnp.where(e == 1.0, idx, VOCAB),
                                axis=-1, keepdims=True))
            o = jax.lax.dot_general(
                e, memt, (((1,), (0,)), ((), ())),
                preferred_element_type=jnp.float32) * rcp
            zq_ref[:, t, :] = o
        tok_ref[...] = jnp.concatenate(toks, axis=1)


@jax.jit
def kernel(z, embeddings, W1, b1, W2, b2, W3, b3, W4, b4):
    bk = z.shape[0] // T

    tok, zq = pl.pallas_call(
        _fused_kernel,
        grid=(NV,),
        in_specs=[
            pl.BlockSpec((VB, T * E), lambda v: (v, 0)),
            pl.BlockSpec((bk, T * E), lambda v: (0, 0)),
            pl.BlockSpec((E, H), lambda v: (0, 0)),
            pl.BlockSpec((1, H), lambda v: (0, 0)),
            pl.BlockSpec((H, H), lambda v: (0, 0)),
            pl.BlockSpec((1, H), lambda v: (0, 0)),
            pl.BlockSpec((H, H), lambda v: (0, 0)),
            pl.BlockSpec((1, H), lambda v: (0, 0)),
            pl.BlockSpec((H, E), lambda v: (0, 0)),
            pl.BlockSpec((1, E), lambda v: (0, 0)),
        ],
        out_specs=[
            pl.BlockSpec((bk, T), lambda v: (0, 0)),
            pl.BlockSpec((bk, T, E), lambda v: (0, 0, 0)),
        ],
        out_shape=[
            jax.ShapeDtypeStruct((bk, T), jnp.int32),
            jax.ShapeDtypeStruct((bk, T, E), jnp.float32),
        ],
        scratch_shapes=[pltpu.VMEM((T, VOCAB, E), jnp.float32)],
    )(embeddings.reshape(VOCAB, T * E), z.reshape(bk, T * E),
      W1, b1.reshape(1, H), W2, b2.reshape(1, H),
      W3, b3.reshape(1, H), W4, b4.reshape(1, E))

    return (tok.reshape(bk * T), zq.reshape(bk * T, E))


# speedup vs baseline: 1.0282x; 1.0282x over previous
import jax
import jax.numpy as jnp
from jax.experimental import pallas as pl
from jax.experimental.pallas import tpu as pltpu

VOCAB = 1024
E = 256
K = 8
T = 4
H = 4 * E
VB = 512
NV = VOCAB // VB
EPS = 1e-5


def _layernorm(x):
    mu = jnp.mean(x, axis=-1, keepdims=True)
    var = jnp.mean((x - mu) ** 2, axis=-1, keepdims=True)
    return (x - mu) * jax.lax.rsqrt(var + EPS)


def _mlp_chain(x, w1_ref, b1_ref, w2_ref, b2_ref, w3_ref, b3_ref,
               w4_ref, b4_ref):
    h = jnp.maximum(
        jnp.dot(x, w1_ref[...], preferred_element_type=jnp.float32)
        + b1_ref[...], 0.0)
    h = jnp.maximum(
        jnp.dot(h, w2_ref[...], preferred_element_type=jnp.float32)
        + b2_ref[...], 0.0)
    h = jnp.maximum(
        jnp.dot(h, w3_ref[...], preferred_element_type=jnp.float32)
        + b3_ref[...], 0.0)
    m = (jnp.dot(h, w4_ref[...], preferred_element_type=jnp.float32)
         + b4_ref[...])
    return _layernorm(m)


def _fused_kernel(emb_ref, z_ref, w1_ref, b1_ref, w2_ref, b2_ref,
                  w3_ref, b3_ref, w4_ref, b4_ref, tok_ref, zq_ref,
                  mem_s, z_s):
    v = pl.program_id(0)
    zw = T * E // NV
    z_s[:, pl.ds(v * zw, zw)] = z_ref[...]
    for t in range(T):
        x = emb_ref[:, t * E:(t + 1) * E]
        mem_s[t, pl.ds(v * VB, VB), :] = _mlp_chain(
            x, w1_ref, b1_ref, w2_ref, b2_ref, w3_ref, b3_ref,
            w4_ref, b4_ref)

    @pl.when(v == NV - 1)
    def _attention():
        toks = []
        for t in range(T):
            q = z_s[:, t * E:(t + 1) * E]
            qn = _layernorm(q) * (E ** -0.5)
            memt = mem_s[t]
            s = jax.lax.dot_general(
                qn, memt, (((1,), (1,)), ((), ())),
                preferred_element_type=jnp.float32)
            mx = jnp.max(s, axis=-1, keepdims=True)
            e = jnp.exp(s - mx)
            rcp = 1.0 / jnp.sum(e, axis=-1, keepdims=True)
            idx = jax.lax.broadcasted_iota(jnp.int32, s.shape, 1)
            toks.append(jnp.min(jnp.where(e == 1.0, idx, VOCAB),
                                axis=-1, keepdims=True))
            o = jax.lax.dot_general(
                e, memt, (((1,), (0,)), ((), ())),
                preferred_element_type=jnp.float32) * rcp
            zq_ref[:, t, :] = o
        tok_ref[...] = jnp.concatenate(toks, axis=1)


@jax.jit
def kernel(z, embeddings, W1, b1, W2, b2, W3, b3, W4, b4):
    bk = z.shape[0] // T

    tok, zq = pl.pallas_call(
        _fused_kernel,
        grid=(NV,),
        in_specs=[
            pl.BlockSpec((VB, T * E), lambda v: (v, 0)),
            pl.BlockSpec((bk, T * E // NV), lambda v: (0, v)),
            pl.BlockSpec((E, H), lambda v: (0, 0)),
            pl.BlockSpec((1, H), lambda v: (0, 0)),
            pl.BlockSpec((H, H), lambda v: (0, 0)),
            pl.BlockSpec((1, H), lambda v: (0, 0)),
            pl.BlockSpec((H, H), lambda v: (0, 0)),
            pl.BlockSpec((1, H), lambda v: (0, 0)),
            pl.BlockSpec((H, E), lambda v: (0, 0)),
            pl.BlockSpec((1, E), lambda v: (0, 0)),
        ],
        out_specs=[
            pl.BlockSpec((bk, T), lambda v: (0, 0)),
            pl.BlockSpec((bk, T, E), lambda v: (0, 0, 0)),
        ],
        out_shape=[
            jax.ShapeDtypeStruct((bk, T), jnp.int32),
            jax.ShapeDtypeStruct((bk, T, E), jnp.float32),
        ],
        scratch_shapes=[pltpu.VMEM((T, VOCAB, E), jnp.float32),
                        pltpu.VMEM((bk, T * E), jnp.float32)],
    )(embeddings.reshape(VOCAB, T * E), z.reshape(bk, T * E),
      W1, b1.reshape(1, H), W2, b2.reshape(1, H),
      W3, b3.reshape(1, H), W4, b4.reshape(1, E))

    return (tok.reshape(bk * T), zq.reshape(bk * T, E))
